# baseline (speedup 1.0000x reference)
import functools

import jax
import jax.numpy as jnp
from jax import lax
from jax.experimental import pallas as pl
from jax.experimental.pallas import tpu as pltpu
from jax.experimental.pallas import tpu_sc as plsc

_NUM_KP = 6
_ROW = 768


def _build_sc_call(B, NC, NS):
    NW = NC * NS
    b_per_w = B // NW
    CH = 128
    n_ch = b_per_w // CH
    mesh = plsc.VectorSubcoreMesh(core_axis_name="c", subcore_axis_name="s")

    @functools.partial(
        pl.kernel,
        mesh=mesh,
        out_type=jax.ShapeDtypeStruct((B, _ROW), jnp.float32),
        scratch_types=[
            pltpu.VMEM((n_ch, CH), jnp.int32),
            pltpu.VMEM((CH, _ROW), jnp.float32),
            pltpu.SemaphoreType.DMA,
        ],
    )
    def sc_gather(table_hbm, idx_hbm, out_hbm, idx_v, rows_v, gsem):
        wid = lax.axis_index("s") * NC + lax.axis_index("c")
        base = wid * b_per_w
        pltpu.sync_copy(idx_hbm.at[wid], idx_v)
        for c in range(n_ch):
            pltpu.async_copy(table_hbm.at[idx_v.at[c]], rows_v, gsem).wait()
            pltpu.sync_copy(rows_v, out_hbm.at[pl.ds(base + c * CH, CH)])

    return sc_gather


def kernel(grip_state, distinction_table, state_table):
    B = grip_state.shape[0]
    num_kp = distinction_table.shape[0]
    info = plsc.get_sparse_core_info()
    NC, NS = info.num_cores, info.num_subcores
    NW = NC * NS

    f = jnp.broadcast_to(distinction_table[None], (2,) + distinction_table.shape)
    s = jnp.broadcast_to(state_table[:, None, :], (2, num_kp, state_table.shape[-1]))
    fused = jnp.concatenate([f, s], axis=-1).reshape(2, _ROW)

    b_per_w = B // NW
    CH = 128
    idx = grip_state.astype(jnp.int32).reshape(NW, b_per_w // CH, CH)

    out = _build_sc_call(B, NC, NS)(fused, idx)
    return out.reshape(B, num_kp, _ROW // num_kp)
